# baseline (speedup 1.0000x reference)
import functools

import jax
import jax.numpy as jnp
from jax import lax
from jax.experimental import pallas as pl
from jax.experimental.pallas import tpu as pltpu
from jax.experimental.pallas import tpu_sc as plsc

PITCH_NUM = 128
BEAT_NUM = 64
EMB = 128

NC = 2
NS = 16
L = 16
NW = NC * NS

BLK = 128
ROWS_BLK = 3 * BLK


@functools.cache
def _build(ntok: int):
    assert ntok % (NW * 2 * BLK) == 0
    tpw = ntok // NW
    nblk = tpw // BLK

    mesh = plsc.VectorSubcoreMesh(core_axis_name="c", subcore_axis_name="s")

    @functools.partial(
        pl.kernel,
        out_type=jax.ShapeDtypeStruct((3 * ntok, EMB), jnp.float32),
        mesh=mesh,
        compiler_params=pltpu.CompilerParams(needs_layout_passes=False),
        scratch_types=[
            pltpu.VMEM((4 * tpw,), jnp.int32),
            pltpu.VMEM((2, 3, BLK), jnp.int32),
            pltpu.VMEM((2, ROWS_BLK, EMB), jnp.float32),
            pltpu.SemaphoreType.DMA,
            pltpu.SemaphoreType.DMA,
            pltpu.SemaphoreType.DMA,
            pltpu.SemaphoreType.DMA,
        ],
    )
    def emb_kernel(tbl_hbm, x_hbm, out_hbm, x_v, idx_v, rows_v, sg0, sg1,
                   so0, so1):
        wid = lax.axis_index("s") * NC + lax.axis_index("c")
        lane = lax.broadcasted_iota(jnp.int32, (L,), 0)
        sg = (sg0, sg1)
        so = (so0, so1)

        pltpu.sync_copy(x_hbm.at[pl.ds(wid * (4 * tpw), 4 * tpw)], x_v)

        def stage(j, b):
            xbase = j * (4 * BLK)
            for g in range(BLK // L):
                t = g * L + lane
                for f, off in ((1, 0), (2, PITCH_NUM),
                               (3, PITCH_NUM + BEAT_NUM)):
                    vals = plsc.load_gather(x_v, [xbase + t * 4 + f]) + off
                    pos = t * 3 + (f - 1)
                    plsc.store_scatter(idx_v.at[b], [pos >> 7, pos & 127],
                                       vals)
            for k in range(3):
                pltpu.async_copy(
                    tbl_hbm.at[idx_v.at[b, k]],
                    rows_v.at[b, pl.ds(k * BLK, BLK)],
                    sg[b],
                )

        def wait_gathers(b):
            pltpu.make_async_copy(
                out_hbm.at[pl.ds(0, ROWS_BLK)], rows_v.at[b], sg[b]
            ).wait()

        def out_row0(j):
            return wid * (3 * tpw) + j * ROWS_BLK

        def issue_out(j, b):
            pltpu.async_copy(
                rows_v.at[b], out_hbm.at[pl.ds(out_row0(j), ROWS_BLK)], so[b]
            )

        def wait_out(j, b):
            pltpu.make_async_copy(
                rows_v.at[b], out_hbm.at[pl.ds(out_row0(j), ROWS_BLK)], so[b]
            ).wait()

        stage(0, 0)
        stage(1, 1)

        def body(jj, carry):
            for b in range(2):
                j = 2 * jj + b
                wait_gathers(b)
                issue_out(j, b)

                @pl.when(j + 2 < nblk)
                def _():
                    wait_out(j, b)
                    stage(j + 2, b)

            return carry

        lax.fori_loop(0, nblk // 2, body, 0)
        wait_out(nblk - 2, 0)
        wait_out(nblk - 1, 1)

    return emb_kernel


def kernel(x, pitch_embedding, beat_embedding, dur_embedding):
    b, s, _ = x.shape
    ntok = b * s
    tbl = jnp.concatenate(
        [pitch_embedding, beat_embedding, dur_embedding], axis=0
    ).astype(jnp.float32)
    x_flat = x.astype(jnp.int32).reshape(-1)
    out = _build(ntok)(tbl, x_flat)
    return out.reshape(b, s, 3 * EMB)


# speedup vs baseline: 1.3587x; 1.3587x over previous
import functools

import jax
import jax.numpy as jnp
from jax import lax
from jax.experimental import pallas as pl
from jax.experimental.pallas import tpu as pltpu
from jax.experimental.pallas import tpu_sc as plsc

PITCH_NUM = 128
BEAT_NUM = 64
EMB = 128

NC = 2
NS = 16
L = 16
NW = NC * NS

SEQ_BLK = 2
IDX_PAD = 128


@functools.cache
def _build(batch: int, seq: int):
    assert batch % (NW * SEQ_BLK) == 0
    blk = SEQ_BLK * seq
    assert blk <= IDX_PAD <= 128
    spw = batch // NW
    tpw = spw * seq
    nblk = spw // SEQ_BLK

    mesh = plsc.VectorSubcoreMesh(core_axis_name="c", subcore_axis_name="s")

    @functools.partial(
        pl.kernel,
        out_type=jax.ShapeDtypeStruct((batch, seq, 3 * EMB), jnp.float32),
        mesh=mesh,
        compiler_params=pltpu.CompilerParams(needs_layout_passes=False),
        scratch_types=[
            pltpu.VMEM((4 * tpw + 4 * IDX_PAD,), jnp.int32),
            pltpu.VMEM((6 * IDX_PAD,), jnp.int32),
            pltpu.VMEM((2, 3, blk, EMB), jnp.float32),
            pltpu.SemaphoreType.DMA,
            pltpu.SemaphoreType.DMA,
            pltpu.SemaphoreType.DMA,
            pltpu.SemaphoreType.DMA,
        ],
    )
    def emb_kernel(tbl_hbm, x_hbm, out_hbm, x_v, idx_v, rows_v, sg0, sg1,
                   so0, so1):
        wid = lax.axis_index("s") * NC + lax.axis_index("c")
        lane = lax.broadcasted_iota(jnp.int32, (L,), 0)
        sg = (sg0, sg1)
        so = (so0, so1)

        pltpu.sync_copy(
            x_hbm.at[pl.ds(wid * (4 * tpw), 4 * tpw)],
            x_v.at[pl.ds(0, 4 * tpw)],
        )

        def stage(j, b):
            xbase = j * (4 * blk)
            for f, off in ((1, 0), (2, PITCH_NUM), (3, PITCH_NUM + BEAT_NUM)):
                ibase = (b * 3 + (f - 1)) * IDX_PAD
                for g in range(IDX_PAD // L):
                    t = g * L + lane
                    vals = plsc.load_gather(x_v, [xbase + t * 4 + f]) + off
                    idx_v[pl.ds(ibase + g * L, L)] = vals
            for k in range(3):
                pltpu.async_copy(
                    tbl_hbm.at[idx_v.at[pl.ds((b * 3 + k) * IDX_PAD, blk)]],
                    rows_v.at[b, k],
                    sg[b],
                )

        def wait_gathers(b):
            for k in range(3):
                pltpu.make_async_copy(
                    tbl_hbm.at[idx_v.at[pl.ds((b * 3 + k) * IDX_PAD, blk)]],
                    rows_v.at[b, k],
                    sg[b],
                ).wait()

        def out_copies(j, b, make_only):
            seq0 = wid * spw + j * SEQ_BLK
            for k in range(3):
                for sb in range(SEQ_BLK):
                    src = rows_v.at[b, k, pl.ds(sb * seq, seq)]
                    dst = out_hbm.at[seq0 + sb, :, pl.ds(k * EMB, EMB)]
                    if make_only:
                        pltpu.make_async_copy(src, dst, so[b]).wait()
                    else:
                        pltpu.async_copy(src, dst, so[b])

        stage(0, 0)
        stage(1, 1)

        def body(jj, carry):
            for b in range(2):
                j = 2 * jj + b
                wait_gathers(b)
                out_copies(j, b, make_only=False)

                @pl.when(j + 2 < nblk)
                def _():
                    out_copies(j, b, make_only=True)
                    stage(j + 2, b)

            return carry

        lax.fori_loop(0, nblk // 2, body, 0)
        out_copies(nblk - 2, 0, make_only=True)
        out_copies(nblk - 1, 1, make_only=True)

    return emb_kernel


def kernel(x, pitch_embedding, beat_embedding, dur_embedding):
    batch, seq, _ = x.shape
    tbl = jnp.concatenate(
        [pitch_embedding, beat_embedding, dur_embedding], axis=0
    ).astype(jnp.float32)
    x_flat = x.astype(jnp.int32).reshape(-1)
    return _build(batch, seq)(tbl, x_flat)


# speedup vs baseline: 3.0250x; 2.2263x over previous
import functools

import jax
import jax.numpy as jnp
from jax import lax
from jax.experimental import pallas as pl
from jax.experimental.pallas import tpu as pltpu
from jax.experimental.pallas import tpu_sc as plsc

PITCH_NUM = 128
BEAT_NUM = 64
EMB = 128

NC = 2
NS = 16
L = 16
NW = NC * NS

SEQ_BLK = 2
IDX_PAD = 128


@functools.cache
def _build(batch: int, seq: int):
    assert batch % (NW * SEQ_BLK) == 0
    blk = SEQ_BLK * seq
    assert blk <= IDX_PAD <= 128
    spw = batch // NW
    tpw = spw * seq
    nblk = spw // SEQ_BLK

    mesh = plsc.VectorSubcoreMesh(core_axis_name="c", subcore_axis_name="s")

    @functools.partial(
        pl.kernel,
        out_type=jax.ShapeDtypeStruct((batch, seq, 3 * EMB), jnp.float32),
        mesh=mesh,
        compiler_params=pltpu.CompilerParams(needs_layout_passes=False),
        scratch_types=[
            pltpu.VMEM((4 * tpw + 4 * IDX_PAD,), jnp.int32),
            pltpu.VMEM((6 * IDX_PAD,), jnp.int32),
            pltpu.VMEM((2, 3, blk, EMB), jnp.float32),
            pltpu.VMEM_SHARED((PITCH_NUM + BEAT_NUM + PITCH_NUM, EMB),
                              jnp.float32),
            pltpu.SemaphoreType.DMA,
            pltpu.SemaphoreType.DMA,
            pltpu.SemaphoreType.DMA,
            pltpu.SemaphoreType.DMA,
        ],
    )
    def emb_kernel(tbl_hbm, x_hbm, out_hbm, x_v, idx_v, rows_v, tbl_sh,
                   sg0, sg1, so0, so1):
        wid = lax.axis_index("s") * NC + lax.axis_index("c")
        lane = lax.broadcasted_iota(jnp.int32, (L,), 0)
        sg = (sg0, sg1)
        so = (so0, so1)

        @pl.when(lax.axis_index("s") == 0)
        def _():
            pltpu.sync_copy(tbl_hbm, tbl_sh)

        pltpu.sync_copy(
            x_hbm.at[pl.ds(wid * (4 * tpw), 4 * tpw)],
            x_v.at[pl.ds(0, 4 * tpw)],
        )
        plsc.subcore_barrier()

        def stage(j, b):
            xbase = j * (4 * blk)
            for f, off in ((1, 0), (2, PITCH_NUM), (3, PITCH_NUM + BEAT_NUM)):
                ibase = (b * 3 + (f - 1)) * IDX_PAD
                for g in range(IDX_PAD // L):
                    t = g * L + lane
                    vals = plsc.load_gather(x_v, [xbase + t * 4 + f]) + off
                    idx_v[pl.ds(ibase + g * L, L)] = vals
            for k in range(3):
                pltpu.async_copy(
                    tbl_sh.at[idx_v.at[pl.ds((b * 3 + k) * IDX_PAD, blk)]],
                    rows_v.at[b, k],
                    sg[b],
                )

        def wait_gathers(b):
            for k in range(3):
                pltpu.make_async_copy(
                    tbl_sh.at[idx_v.at[pl.ds((b * 3 + k) * IDX_PAD, blk)]],
                    rows_v.at[b, k],
                    sg[b],
                ).wait()

        def out_copies(j, b, make_only):
            seq0 = wid * spw + j * SEQ_BLK
            for k in range(3):
                for sb in range(SEQ_BLK):
                    src = rows_v.at[b, k, pl.ds(sb * seq, seq)]
                    dst = out_hbm.at[seq0 + sb, :, pl.ds(k * EMB, EMB)]
                    if make_only:
                        pltpu.make_async_copy(src, dst, so[b]).wait()
                    else:
                        pltpu.async_copy(src, dst, so[b])

        stage(0, 0)
        stage(1, 1)

        def body(jj, carry):
            for b in range(2):
                j = 2 * jj + b
                wait_gathers(b)
                out_copies(j, b, make_only=False)

                @pl.when(j + 2 < nblk)
                def _():
                    out_copies(j, b, make_only=True)
                    stage(j + 2, b)

            return carry

        lax.fori_loop(0, nblk // 2, body, 0)
        out_copies(nblk - 2, 0, make_only=True)
        out_copies(nblk - 1, 1, make_only=True)

    return emb_kernel


def kernel(x, pitch_embedding, beat_embedding, dur_embedding):
    batch, seq, _ = x.shape
    tbl = jnp.concatenate(
        [pitch_embedding, beat_embedding, dur_embedding], axis=0
    ).astype(jnp.float32)
    x_flat = x.astype(jnp.int32).reshape(-1)
    return _build(batch, seq)(tbl, x_flat)


# speedup vs baseline: 9.5400x; 3.1538x over previous
import functools

import jax
import jax.numpy as jnp
from jax import lax
from jax.experimental import pallas as pl
from jax.experimental.pallas import tpu as pltpu
from jax.experimental.pallas import tpu_sc as plsc

EMB = 128

NC = 2
NS = 16
NW = NC * NS


@functools.cache
def _build(batch: int, seq: int, pitch_num: int, beat_num: int,
           dur_num: int):
    assert batch % NW == 0
    bpw = batch // NW

    mesh = plsc.VectorSubcoreMesh(core_axis_name="c", subcore_axis_name="s")

    @functools.partial(
        pl.kernel,
        out_type=jax.ShapeDtypeStruct((seq, batch, 3 * EMB), jnp.float32),
        mesh=mesh,
        compiler_params=pltpu.CompilerParams(needs_layout_passes=False),
        scratch_types=[
            pltpu.VMEM((seq, 4, bpw), jnp.int32),
            pltpu.VMEM((2, 3, bpw, EMB), jnp.float32),
            pltpu.VMEM_SHARED((pitch_num, EMB), jnp.float32),
            pltpu.VMEM_SHARED((beat_num, EMB), jnp.float32),
            pltpu.VMEM_SHARED((dur_num, EMB), jnp.float32),
            pltpu.SemaphoreType.DMA,
            pltpu.SemaphoreType.DMA,
            pltpu.SemaphoreType.DMA,
            pltpu.SemaphoreType.DMA,
        ],
    )
    def emb_kernel(pitch_hbm, beat_hbm, dur_hbm, xt_hbm, out_hbm,
                   x_v, rows_v, tp_sh, tb_sh, td_sh, sg0, sg1, so0, so1):
        wid = lax.axis_index("s") * NC + lax.axis_index("c")
        b0 = wid * bpw
        sg = (sg0, sg1)
        so = (so0, so1)
        tables = (tp_sh, tb_sh, td_sh)

        @pl.when(lax.axis_index("s") == 0)
        def _():
            pltpu.sync_copy(pitch_hbm, tp_sh)
            pltpu.sync_copy(beat_hbm, tb_sh)
            pltpu.sync_copy(dur_hbm, td_sh)

        pltpu.sync_copy(xt_hbm.at[:, :, pl.ds(b0, bpw)], x_v)
        plsc.subcore_barrier()

        def gathers(si, b, make_only):
            for k in range(3):
                cp = pltpu.make_async_copy(
                    tables[k].at[x_v.at[si, k + 1]], rows_v.at[b, k], sg[b]
                )
                if make_only:
                    cp.wait()
                else:
                    cp.start()

        def out_copies(si, b, make_only):
            for k in range(3):
                cp = pltpu.make_async_copy(
                    rows_v.at[b, k],
                    out_hbm.at[si, pl.ds(b0, bpw), pl.ds(k * EMB, EMB)],
                    so[b],
                )
                if make_only:
                    cp.wait()
                else:
                    cp.start()

        gathers(0, 0, make_only=False)
        gathers(1, 1, make_only=False)

        def body(jj, carry):
            for b in range(2):
                si = 2 * jj + b
                gathers(si, b, make_only=True)
                out_copies(si, b, make_only=False)

                @pl.when(si + 2 < seq)
                def _():
                    out_copies(si, b, make_only=True)
                    gathers(si + 2, b, make_only=False)

            return carry

        lax.fori_loop(0, seq // 2, body, 0)
        out_copies(seq - 2, 0, make_only=True)
        out_copies(seq - 1, 1, make_only=True)

    return emb_kernel


def kernel(x, pitch_embedding, beat_embedding, dur_embedding):
    batch, seq, _ = x.shape
    xt = jnp.transpose(x.astype(jnp.int32), (1, 2, 0))
    fn = _build(batch, seq, pitch_embedding.shape[0],
                beat_embedding.shape[0], dur_embedding.shape[0])
    res = fn(pitch_embedding.astype(jnp.float32),
             beat_embedding.astype(jnp.float32),
             dur_embedding.astype(jnp.float32), xt)
    return jnp.transpose(res, (1, 0, 2))
